# Initial kernel scaffold; baseline (speedup 1.0000x reference)
import jax
import jax.numpy as jnp
from jax.experimental import pallas as pl


def kernel(x_user, x_item, edge_index_clicks, edge_index_clicked_by, Wp_user, bp_user, Wp_item, bp_item, W0_c, b0_c, W0_cb, b0_cb, W1_c, b1_c, W1_cb, b1_cb, W_out, b_out):
    raise NotImplementedError("write your pallas kernel here")


# speedup vs baseline: 4.0674x; 4.0674x over previous
import functools

import jax
import jax.numpy as jnp
from jax import lax
from jax.experimental import pallas as pl
from jax.experimental.pallas import tpu as pltpu
from jax.experimental.pallas import tpu_sc as plsc

NU = 10000
NI = 10000
E = 320000
D = 128
H = 128
O = 64

NC = 2
NS = 16
NW = NC * NS
CHUNK = 80
EPW = E // NW
NITER = EPW // CHUNK
EPT = E // NS
DITER = EPT // CHUNK
NP = 10240
RPT = NP // NS
RB = 1000


def _proj(x, w, b):
    def body(x_ref, w_ref, b_ref, o_ref):
        o_ref[...] = lax.dot_general(
            x_ref[...], w_ref[...], (((1,), (1,)), ((), ())),
            preferred_element_type=jnp.float32) + b_ref[...]

    return pl.pallas_call(
        body,
        grid=(NI // RB,),
        in_specs=[pl.BlockSpec((RB, D), lambda i: (i, 0)),
                  pl.BlockSpec((H, D), lambda i: (0, 0)),
                  pl.BlockSpec((1, H), lambda i: (0, 0))],
        out_specs=pl.BlockSpec((RB, H), lambda i: (i, 0)),
        out_shape=jax.ShapeDtypeStruct((NI, H), jnp.float32),
    )(x, w, b)


def _mid(acc, deg, w, b):
    def body(a_ref, d_ref, w_ref, b_ref, o_ref):
        p = a_ref[0] + a_ref[1]
        dg = d_ref[...]
        mean = p / jnp.maximum(dg, 1.0)
        h = lax.dot_general(mean, w_ref[...], (((1,), (1,)), ((), ())),
                            preferred_element_type=jnp.float32) + b_ref[...]
        h = jnp.where(dg > 0, h, 0.0)
        o_ref[...] = jnp.where(h >= 0, h, 0.01 * h)

    return pl.pallas_call(
        body,
        grid=(NU // RB,),
        in_specs=[pl.BlockSpec((2, RB, H), lambda i: (0, i, 0)),
                  pl.BlockSpec((RB, 1), lambda i: (i, 0)),
                  pl.BlockSpec((H, H), lambda i: (0, 0)),
                  pl.BlockSpec((1, H), lambda i: (0, 0))],
        out_specs=pl.BlockSpec((RB, H), lambda i: (i, 0)),
        out_shape=jax.ShapeDtypeStruct((NU, H), jnp.float32),
    )(acc, deg, w, b)


def _final(acc, deg, w1, b1, wo, bo):
    def body(a_ref, d_ref, w1_ref, b1_ref, wo_ref, bo_ref, o_ref):
        p = a_ref[0] + a_ref[1]
        dg = d_ref[...]
        mean = p / jnp.maximum(dg, 1.0)
        h = lax.dot_general(mean, w1_ref[...], (((1,), (1,)), ((), ())),
                            preferred_element_type=jnp.float32) + b1_ref[...]
        h = jnp.where(dg > 0, h, 0.0)
        o_ref[...] = lax.dot_general(h, wo_ref[...], (((1,), (1,)), ((), ())),
                                     preferred_element_type=jnp.float32) + bo_ref[...]

    return pl.pallas_call(
        body,
        grid=(NI // RB,),
        in_specs=[pl.BlockSpec((2, RB, H), lambda i: (0, i, 0)),
                  pl.BlockSpec((RB, 1), lambda i: (i, 0)),
                  pl.BlockSpec((H, H), lambda i: (0, 0)),
                  pl.BlockSpec((1, H), lambda i: (0, 0)),
                  pl.BlockSpec((O, H), lambda i: (0, 0)),
                  pl.BlockSpec((1, O), lambda i: (0, 0))],
        out_specs=pl.BlockSpec((RB, O), lambda i: (i, 0)),
        out_shape=jax.ShapeDtypeStruct((NI, O), jnp.float32),
    )(acc, deg, w1, b1, wo, bo)


def _make_deg():
    mesh = plsc.VectorSubcoreMesh(core_axis_name="c", subcore_axis_name="s")

    @functools.partial(
        pl.kernel,
        mesh=mesh,
        out_type=jax.ShapeDtypeStruct((NC, NP, H), jnp.float32),
        scratch_types=[
            pltpu.VMEM((CHUNK,), jnp.int32),
            pltpu.VMEM((CHUNK, H), jnp.float32),
            pltpu.VMEM_SHARED((NP, H), jnp.float32),
        ],
    )
    def deg(dstb_hbm, dstc_hbm, ones_hbm, zero_hbm, out_hbm,
            dst_v, ones_v, acc_sh):
        cid = lax.axis_index("c")
        sid = lax.axis_index("s")
        pltpu.sync_copy(zero_hbm.at[pl.ds(sid * RPT, RPT)],
                        acc_sh.at[pl.ds(sid * RPT, RPT)])
        pltpu.sync_copy(ones_hbm, ones_v)
        plsc.subcore_barrier()

        base = sid * EPT

        def run(dst_hbm):
            def body(g, carry):
                off = base + g * CHUNK
                pltpu.sync_copy(dst_hbm.at[pl.ds(off, CHUNK)], dst_v)
                pltpu.sync_copy(ones_v, acc_sh.at[dst_v], add=True)
                return carry

            lax.fori_loop(0, DITER, body, 0)

        @pl.when(cid == 0)
        def _():
            run(dstb_hbm)

        @pl.when(cid == 1)
        def _():
            run(dstc_hbm)

        plsc.subcore_barrier()
        pltpu.sync_copy(acc_sh.at[pl.ds(sid * RPT, RPT)],
                        out_hbm.at[cid, pl.ds(sid * RPT, RPT)])

    return deg


def _make_segsum():
    mesh = plsc.VectorSubcoreMesh(core_axis_name="c", subcore_axis_name="s")

    @functools.partial(
        pl.kernel,
        mesh=mesh,
        out_type=jax.ShapeDtypeStruct((NC, NP, H), jnp.float32),
        scratch_types=[
            pltpu.VMEM((CHUNK,), jnp.int32),
            pltpu.VMEM((CHUNK,), jnp.int32),
            pltpu.VMEM((CHUNK, H), jnp.float32),
            pltpu.VMEM_SHARED((NP, H), jnp.float32),
            pltpu.SemaphoreType.DMA,
        ],
    )
    def seg(table_hbm, src_hbm, dst_hbm, zero_hbm, out_hbm,
            src_v, dst_v, rows_v, acc_sh, sem):
        cid = lax.axis_index("c")
        sid = lax.axis_index("s")
        pltpu.sync_copy(zero_hbm.at[pl.ds(sid * RPT, RPT)],
                        acc_sh.at[pl.ds(sid * RPT, RPT)])
        plsc.subcore_barrier()

        wid = sid * NC + cid
        base = wid * EPW

        def body(g, carry):
            off = base + g * CHUNK
            pltpu.sync_copy(src_hbm.at[pl.ds(off, CHUNK)], src_v)
            pltpu.sync_copy(dst_hbm.at[pl.ds(off, CHUNK)], dst_v)
            pltpu.async_copy(table_hbm.at[src_v], rows_v, sem).wait()
            pltpu.sync_copy(rows_v, acc_sh.at[dst_v], add=True)
            return carry

        lax.fori_loop(0, NITER, body, 0)
        plsc.subcore_barrier()
        pltpu.sync_copy(acc_sh.at[pl.ds(sid * RPT, RPT)],
                        out_hbm.at[cid, pl.ds(sid * RPT, RPT)])

    return seg


_deg_pass = _make_deg()
_segsum_u = _make_segsum()
_segsum_i = _make_segsum()


def kernel(x_user, x_item, edge_index_clicks, edge_index_clicked_by,
           Wp_user, bp_user, Wp_item, bp_item,
           W0_c, b0_c, W0_cb, b0_cb,
           W1_c, b1_c, W1_cb, b1_cb,
           W_out, b_out):
    src_c = edge_index_clicks[0].astype(jnp.int32)
    dst_c = edge_index_clicks[1].astype(jnp.int32)
    src_b = edge_index_clicked_by[0].astype(jnp.int32)
    dst_b = edge_index_clicked_by[1].astype(jnp.int32)
    zeros = jnp.zeros((NP, H), jnp.float32)
    ones = jnp.ones((CHUNK, H), jnp.float32)

    degf = _deg_pass(dst_b, dst_c, ones, zeros)
    deg_b = degf[0, :, 0:1]
    deg_c = degf[1, :, 0:1]

    hi = _proj(x_item, Wp_item, bp_item.reshape(1, H))
    acc_b = _segsum_u(hi, src_b, dst_b, zeros)
    hu1 = _mid(acc_b, deg_b, W0_cb, b0_cb.reshape(1, H))
    acc_c = _segsum_i(hu1, src_c, dst_c, zeros)
    return _final(acc_c, deg_c, W1_c, b1_c.reshape(1, H),
                  W_out, b_out.reshape(1, O))
